# baseline (speedup 1.0000x reference)
import functools
import math

import jax
import jax.numpy as jnp
from jax import lax
from jax.experimental import pallas as pl
from jax.experimental.pallas import tpu as pltpu
from jax.experimental.pallas import tpu_sc as plsc

N = 10000
E = 320000
F_IN = 128
H = 64
G = 16
LAYERS = 4
EPS = 1e-05

NC = 2
NS = 16
LN = 16
NW = NC * NS

NP = 10240
NPT = NP // NS
EPAD = 327680
ER = EPAD // 128
RPW = ER // NW
CR = 4
CE = CR * 128
NCHUNK = RPW // CR

_BN_SCALE = 1.0 / math.sqrt(1.0 + EPS)


def _wid():
    return lax.axis_index("c") * NS + lax.axis_index("s")


def _mesh():
    return plsc.VectorSubcoreMesh(core_axis_name="c", subcore_axis_name="s")


def _sc_params():
    return pltpu.CompilerParams(
        needs_layout_passes=False, use_tc_tiling_on_sc=False)


def _deg_body(col_hbm, w_hbm, out_hbm, colb, wb, zb, degacc):
    c = lax.axis_index("c")
    s = lax.axis_index("s")
    wid = c * NS + s

    def zfill(i, _):
        zb[pl.ds(i * LN, LN)] = jnp.zeros((LN,), jnp.float32)
        return 0

    lax.fori_loop(0, NPT // LN, zfill, 0)
    pltpu.sync_copy(zb, degacc.at[pl.ds(s * NPT, NPT)])
    plsc.subcore_barrier()

    def chunk(i, _):
        base = wid * RPW + i * CR
        pltpu.sync_copy(col_hbm.at[pl.ds(base, CR)], colb)
        pltpu.sync_copy(w_hbm.at[pl.ds(base, CR)], wb)
        for r in range(CR):
            pltpu.sync_copy(wb.at[r], degacc.at[colb.at[r]], add=True)
        return 0

    lax.fori_loop(0, NCHUNK, chunk, 0)
    plsc.subcore_barrier()
    pltpu.sync_copy(degacc.at[pl.ds(s * NPT, NPT)], zb)
    pltpu.sync_copy(zb, out_hbm.at[c, pl.ds(s * NPT, NPT)])


def _sc_deg(colp, wp):
    f = pl.kernel(
        _deg_body,
        compiler_params=_sc_params(),
        out_type=jax.ShapeDtypeStruct((NC, NP), jnp.float32),
        mesh=_mesh(),
        scratch_types=[
            pltpu.VMEM((CR, 128), jnp.int32),
            pltpu.VMEM((CR, 128), jnp.float32),
            pltpu.VMEM((NPT,), jnp.float32),
            pltpu.VMEM_SHARED((NP,), jnp.float32),
        ],
    )
    return f(colp, wp)


def _norm_body(row_hbm, col_hbm, w_hbm, dinv_hbm, out_hbm, rb, cb, wb, nb, dinv_v):
    wid = _wid()
    pltpu.sync_copy(dinv_hbm, dinv_v)

    def chunk(i, _):
        base = wid * RPW + i * CR
        pltpu.sync_copy(row_hbm.at[pl.ds(base, CR)], rb)
        pltpu.sync_copy(col_hbm.at[pl.ds(base, CR)], cb)
        pltpu.sync_copy(w_hbm.at[pl.ds(base, CR)], wb)
        for g in range(CE // LN):
            r = g // (128 // LN)
            off = (g % (128 // LN)) * LN
            rv = rb[r, pl.ds(off, LN)]
            cv = cb[r, pl.ds(off, LN)]
            wv = wb[r, pl.ds(off, LN)]
            dr = plsc.load_gather(dinv_v, [rv])
            dc = plsc.load_gather(dinv_v, [cv])
            nb[pl.ds(g * LN, LN)] = dr * wv * dc
        pltpu.sync_copy(nb, out_hbm.at[pl.ds(base * 128, CE)])
        return 0

    lax.fori_loop(0, NCHUNK, chunk, 0)


def _sc_norm(rowp, colp, wp, dinv_flat):
    f = pl.kernel(
        _norm_body,
        compiler_params=_sc_params(),
        out_type=jax.ShapeDtypeStruct((EPAD,), jnp.float32),
        mesh=_mesh(),
        scratch_types=[
            pltpu.VMEM((CR, 128), jnp.int32),
            pltpu.VMEM((CR, 128), jnp.int32),
            pltpu.VMEM((CR, 128), jnp.float32),
            pltpu.VMEM((CE,), jnp.float32),
            pltpu.VMEM((NP,), jnp.float32),
        ],
    )
    return f(rowp, colp, wp, dinv_flat)


def _msg_body(xw_hbm, row_hbm, col_hbm, nexp_hbm, out_hbm,
              rowb0, colb0, nxb0, msg0, rowb1, colb1, nxb1, msg1,
              zb, acc, semg0, semg1):
    c = lax.axis_index("c")
    s = lax.axis_index("s")
    wid = c * NS + s

    def zfill(i, _):
        for q in range(H // LN):
            zb[i, pl.ds(q * LN, LN)] = jnp.zeros((LN,), jnp.float32)
        return 0

    lax.fori_loop(0, 64, zfill, 0)
    for t in range(NPT // 64):
        pltpu.sync_copy(zb, acc.at[pl.ds(s * NPT + t * 64, 64)])
    plsc.subcore_barrier()

    def fetch(rowb, colb, nxb, msg, semg, ci):
        base = wid * RPW + ci * CR
        pltpu.sync_copy(row_hbm.at[pl.ds(base, CR)], rowb)
        pltpu.sync_copy(col_hbm.at[pl.ds(base, CR)], colb)
        pltpu.async_copy(nexp_hbm.at[pl.ds(base * 128, CE)], nxb, semg)
        for r in range(CR):
            pltpu.async_copy(
                xw_hbm.at[rowb.at[r]], msg.at[pl.ds(r * 128, 128)], semg)

    def drain(rowb, nxb, msg, semg):
        pltpu.make_async_copy(
            nexp_hbm.at[pl.ds(0, CE)], nxb, semg).wait()
        for r in range(CR):
            pltpu.make_async_copy(
                xw_hbm.at[rowb.at[r]], msg.at[pl.ds(r * 128, 128)], semg).wait()

    def scale(nxb, msg):
        def group(g, _):
            e0 = g * 4
            for k in range(4):
                nv = nxb[e0 + k, pl.ds(0, LN)]
                for q in range(H // LN):
                    sl = pl.ds(q * LN, LN)
                    msg[e0 + k, sl] = msg[e0 + k, sl] * nv
            return 0

        lax.fori_loop(0, CE // 4, group, 0)

    def scatter(colb, msg):
        for r in range(CR):
            pltpu.sync_copy(msg.at[pl.ds(r * 128, 128)], acc.at[colb.at[r]], add=True)

    fetch(rowb0, colb0, nxb0, msg0, semg0, 0)

    def pipe(i, _):
        fetch(rowb1, colb1, nxb1, msg1, semg1, 2 * i + 1)
        drain(rowb0, nxb0, msg0, semg0)
        scale(nxb0, msg0)
        scatter(colb0, msg0)
        fetch(rowb0, colb0, nxb0, msg0, semg0, lax.rem(2 * i + 2, NCHUNK))
        drain(rowb1, nxb1, msg1, semg1)
        scale(nxb1, msg1)
        scatter(colb1, msg1)
        return 0

    lax.fori_loop(0, NCHUNK // 2, pipe, 0)
    drain(rowb0, nxb0, msg0, semg0)
    plsc.subcore_barrier()
    for t in range(NPT // 64):
        r0 = s * NPT + t * 64
        pltpu.sync_copy(acc.at[pl.ds(r0, 64)], zb)
        pltpu.sync_copy(zb, out_hbm.at[c, pl.ds(r0, 64)])


def _sc_msg(xw, rowp, colp, nexp):
    buf = lambda: [
        pltpu.VMEM((CR, 128), jnp.int32),
        pltpu.VMEM((CR, 128), jnp.int32),
        pltpu.VMEM((CE, LN), jnp.float32),
        pltpu.VMEM((CE, H), jnp.float32),
    ]
    f = pl.kernel(
        _msg_body,
        compiler_params=_sc_params(),
        out_type=jax.ShapeDtypeStruct((NC, NP, H), jnp.float32),
        mesh=_mesh(),
        scratch_types=buf() + buf() + [
            pltpu.VMEM((64, H), jnp.float32),
            pltpu.VMEM_SHARED((NP, H), jnp.float32),
            pltpu.SemaphoreType.DMA,
            pltpu.SemaphoreType.DMA,
        ],
    )
    return f(xw, rowp, colp, nexp)


def _tc_normexp(norm_flat):
    BLK = EPAD // 16

    def body(n_ref, o_ref):
        o_ref[...] = jnp.broadcast_to(n_ref[...], (BLK, LN))

    return pl.pallas_call(
        body,
        grid=(16,),
        in_specs=[pl.BlockSpec((BLK, 1), lambda g: (g, 0))],
        out_specs=pl.BlockSpec((BLK, LN), lambda g: (g, 0)),
        out_shape=jax.ShapeDtypeStruct((EPAD, LN), jnp.float32),
    )(norm_flat.reshape(EPAD, 1))


def _tc_matmul(h, w):
    def body(h_ref, w_ref, o_ref):
        o_ref[...] = jnp.dot(h_ref[...], w_ref[...],
                             preferred_element_type=jnp.float32)

    return pl.pallas_call(
        body,
        out_shape=jax.ShapeDtypeStruct((h.shape[0], w.shape[1]), jnp.float32),
    )(h, w)


def _tc_dinv(degp_t):
    def body(d_ref, o_ref):
        deg = d_ref[:, 0:1] + d_ref[:, 1:2] + 1.0
        o_ref[...] = jnp.where(deg > 0, lax.rsqrt(deg), 0.0)

    return pl.pallas_call(
        body,
        out_shape=jax.ShapeDtypeStruct((NP, 1), jnp.float32),
    )(degp_t)


def _tc_post(accp, xw, dinv_col, b2, g2, bt2, res):
    def body(*refs):
        if res is not None:
            a_ref, xw_ref, d_ref, b_ref, g_ref, bt_ref, r_ref, o_ref = refs
        else:
            a_ref, xw_ref, d_ref, b_ref, g_ref, bt_ref, o_ref = refs
        agg = a_ref[0] + a_ref[1]
        agg = agg[:N, :]
        d = d_ref[:N, :]
        xwv = xw_ref[...]
        t = agg + d * d * xwv + b_ref[...]
        t = t * (g_ref[...] * _BN_SCALE) + bt_ref[...]
        t = jnp.maximum(t, 0.0)
        if res is not None:
            t = t + r_ref[...]
        o_ref[...] = t

    args = [accp, xw, dinv_col, b2, g2, bt2]
    if res is not None:
        args.append(res)
    return pl.pallas_call(
        body,
        out_shape=jax.ShapeDtypeStruct((N, H), jnp.float32),
    )(*args)


def _tc_readout(h, batch_col, p):
    def body(h_ref, b_ref, aw1, ab1, aw2, ab2, mw1, mb1, mw2, mb2, mw3, mb3,
             o_ref):
        hv = h_ref[...]
        bcol = b_ref[...]
        gids = lax.broadcasted_iota(jnp.int32, (N, G), 1)
        M = bcol == gids
        Mf = M.astype(jnp.float32)
        dot = functools.partial(lax.dot_general,
                                dimension_numbers=(((0,), (0,)), ((), ())),
                                precision=lax.Precision.HIGHEST,
                                preferred_element_type=jnp.float32)
        ones_col = jnp.ones((N, 1), jnp.float32)
        cnt = dot(Mf, ones_col)
        x_add = dot(Mf, hv)
        x_mean = x_add / jnp.maximum(cnt, 1.0)
        a1 = jnp.tanh(jnp.dot(hv, aw1[...], preferred_element_type=jnp.float32)
                      + ab1[...])
        attn = jnp.dot(a1, aw2[...], preferred_element_type=jnp.float32) \
            + ab2[...]
        am = jnp.max(jnp.where(M, attn, -jnp.inf), axis=0, keepdims=True)
        mb = jnp.sum(jnp.where(M, am, 0.0), axis=1, keepdims=True)
        e = jnp.exp(attn - mb)
        denom = dot(Mf, e)
        den_node = jnp.dot(Mf, denom, precision=lax.Precision.HIGHEST,
                           preferred_element_type=jnp.float32)
        wnode = e / den_node
        x_attn = dot(Mf, hv * wnode)
        comb = x_mean + 0.5 * x_attn + 0.1 * x_add
        z = jnp.maximum(jnp.dot(comb, mw1[...],
                                preferred_element_type=jnp.float32)
                        + mb1[...], 0.0)
        z = jnp.maximum(jnp.dot(z, mw2[...],
                                preferred_element_type=jnp.float32)
                        + mb2[...], 0.0)
        o_ref[...] = jnp.dot(z, mw3[...],
                             preferred_element_type=jnp.float32) + mb3[...]

    return pl.pallas_call(
        body,
        out_shape=jax.ShapeDtypeStruct((G, 1), jnp.float32),
    )(h, batch_col,
      p["attnW1"], p["attnb1"].reshape(1, -1),
      p["attnW2"], p["attnb2"].reshape(1, -1),
      p["mlpW1"], p["mlpb1"].reshape(1, -1),
      p["mlpW2"], p["mlpb2"].reshape(1, -1),
      p["mlpW3"], p["mlpb3"].reshape(1, -1))


def kernel(x, edge_index, edge_weight, batch, params):
    row = edge_index[0]
    col = edge_index[1]
    padn = EPAD - E
    rowp = jnp.concatenate([row, jnp.zeros((padn,), row.dtype)]).reshape(ER, 128)
    colp = jnp.concatenate([col, jnp.zeros((padn,), col.dtype)]).reshape(ER, 128)
    wp = jnp.concatenate(
        [edge_weight, jnp.zeros((padn,), edge_weight.dtype)]).reshape(ER, 128)

    degp = _sc_deg(colp, wp)
    dinv_col = _tc_dinv(degp.T)
    norm_flat = _sc_norm(rowp, colp, wp, dinv_col.reshape(NP))
    nexp = _tc_normexp(norm_flat)

    h = None
    for i in range(LAYERS):
        inp = x if i == 0 else h
        xw = _tc_matmul(inp, params["convW"][i])
        accp = _sc_msg(xw, rowp, colp, nexp)
        h = _tc_post(
            accp, xw, dinv_col,
            params["convb"][i].reshape(1, H),
            params["bn_g"][i].reshape(1, H),
            params["bn_b"][i].reshape(1, H),
            res=(h if i > 0 else None),
        )

    return _tc_readout(h, batch.reshape(N, 1), params)


# speedup vs baseline: 1.1460x; 1.1460x over previous
import functools
import math

import jax
import jax.numpy as jnp
from jax import lax
from jax.experimental import pallas as pl
from jax.experimental.pallas import tpu as pltpu
from jax.experimental.pallas import tpu_sc as plsc

N = 10000
E = 320000
F_IN = 128
H = 64
G = 16
LAYERS = 4
EPS = 1e-05

NC = 2
NS = 16
LN = 16
NW = NC * NS

NP = 10240
NPT = NP // NS
EPAD = 327680
ER = EPAD // 128
RPW = ER // NW
CR = 4
CE = CR * 128
NCHUNK = RPW // CR
MCR = 2
MCE = MCR * 128
MNCHUNK = RPW // MCR

_BN_SCALE = 1.0 / math.sqrt(1.0 + EPS)


def _wid():
    return lax.axis_index("c") * NS + lax.axis_index("s")


def _mesh():
    return plsc.VectorSubcoreMesh(core_axis_name="c", subcore_axis_name="s")


def _sc_params():
    return pltpu.CompilerParams(
        needs_layout_passes=False, use_tc_tiling_on_sc=False)


def _deg_body(col_hbm, w_hbm, out_hbm, colb, wb, zb, degacc):
    c = lax.axis_index("c")
    s = lax.axis_index("s")
    wid = c * NS + s

    def zfill(i, _):
        zb[pl.ds(i * LN, LN)] = jnp.zeros((LN,), jnp.float32)
        return 0

    lax.fori_loop(0, NPT // LN, zfill, 0)
    pltpu.sync_copy(zb, degacc.at[pl.ds(s * NPT, NPT)])
    plsc.subcore_barrier()

    def chunk(i, _):
        base = wid * RPW + i * CR
        pltpu.sync_copy(col_hbm.at[pl.ds(base, CR)], colb)
        pltpu.sync_copy(w_hbm.at[pl.ds(base, CR)], wb)
        for r in range(CR):
            pltpu.sync_copy(wb.at[r], degacc.at[colb.at[r]], add=True)
        return 0

    lax.fori_loop(0, NCHUNK, chunk, 0)
    plsc.subcore_barrier()
    pltpu.sync_copy(degacc.at[pl.ds(s * NPT, NPT)], zb)
    pltpu.sync_copy(zb, out_hbm.at[c, pl.ds(s * NPT, NPT)])


def _sc_deg(colp, wp):
    f = pl.kernel(
        _deg_body,
        compiler_params=_sc_params(),
        out_type=jax.ShapeDtypeStruct((NC, NP), jnp.float32),
        mesh=_mesh(),
        scratch_types=[
            pltpu.VMEM((CR, 128), jnp.int32),
            pltpu.VMEM((CR, 128), jnp.float32),
            pltpu.VMEM((NPT,), jnp.float32),
            pltpu.VMEM_SHARED((NP,), jnp.float32),
        ],
    )
    return f(colp, wp)


def _norm_body(row_hbm, col_hbm, w_hbm, dinv_hbm, out_hbm, rb, cb, wb, nb, dinv_v):
    wid = _wid()
    pltpu.sync_copy(dinv_hbm, dinv_v)

    def chunk(i, _):
        base = wid * RPW + i * CR
        pltpu.sync_copy(row_hbm.at[pl.ds(base, CR)], rb)
        pltpu.sync_copy(col_hbm.at[pl.ds(base, CR)], cb)
        pltpu.sync_copy(w_hbm.at[pl.ds(base, CR)], wb)
        for g in range(CE // LN):
            r = g // (128 // LN)
            off = (g % (128 // LN)) * LN
            rv = rb[r, pl.ds(off, LN)]
            cv = cb[r, pl.ds(off, LN)]
            wv = wb[r, pl.ds(off, LN)]
            dr = plsc.load_gather(dinv_v, [rv])
            dc = plsc.load_gather(dinv_v, [cv])
            nb[pl.ds(g * LN, LN)] = dr * wv * dc
        pltpu.sync_copy(nb, out_hbm.at[pl.ds(base * 128, CE)])
        return 0

    lax.fori_loop(0, NCHUNK, chunk, 0)


def _sc_norm(rowp, colp, wp, dinv_flat):
    f = pl.kernel(
        _norm_body,
        compiler_params=_sc_params(),
        out_type=jax.ShapeDtypeStruct((EPAD,), jnp.float32),
        mesh=_mesh(),
        scratch_types=[
            pltpu.VMEM((CR, 128), jnp.int32),
            pltpu.VMEM((CR, 128), jnp.int32),
            pltpu.VMEM((CR, 128), jnp.float32),
            pltpu.VMEM((CE,), jnp.float32),
            pltpu.VMEM((NP,), jnp.float32),
        ],
    )
    return f(rowp, colp, wp, dinv_flat)


def _msg_body(xw_hbm, row_hbm, col_hbm, norm_hbm, out_hbm,
              rowb0, colb0, nfb0, msg0, rowb1, colb1, nfb1, msg1,
              zb, acc, xwt, semg0, semg1):
    c = lax.axis_index("c")
    s = lax.axis_index("s")
    wid = c * NS + s

    def zfill(i, _):
        for q in range(H // LN):
            zb[i, pl.ds(q * LN, LN)] = jnp.zeros((LN,), jnp.float32)
        return 0

    lax.fori_loop(0, 32, zfill, 0)
    pltpu.sync_copy(xw_hbm.at[pl.ds(s * (N // NS), N // NS)],
                    xwt.at[pl.ds(s * (N // NS), N // NS)])
    for t in range(NPT // 32):
        pltpu.sync_copy(zb, acc.at[pl.ds(s * NPT + t * 32, 32)])
    plsc.subcore_barrier()

    def fetch(rowb, colb, nfb, msg, semg, ci):
        base = wid * RPW + ci * MCR
        pltpu.sync_copy(row_hbm.at[pl.ds(base, MCR)], rowb)
        pltpu.sync_copy(col_hbm.at[pl.ds(base, MCR)], colb)
        pltpu.sync_copy(norm_hbm.at[pl.ds(base * 128, MCE)], nfb)
        for r in range(MCR):
            pltpu.async_copy(
                xwt.at[rowb.at[r]], msg.at[pl.ds(r * 128, 128)], semg)

    def drain(rowb, msg, semg):
        for r in range(MCR):
            pltpu.make_async_copy(
                xwt.at[rowb.at[r]], msg.at[pl.ds(r * 128, 128)], semg).wait()

    def scale(nfb, msg):
        def group(g, _):
            e0 = g * LN
            nvg = nfb[pl.ds(e0, LN)]
            for k in range(LN):
                nv = jnp.full((LN,), nvg[k])
                for q in range(H // LN):
                    sl = pl.ds(q * LN, LN)
                    msg[e0 + k, sl] = msg[e0 + k, sl] * nv
            return 0

        lax.fori_loop(0, MCE // LN, group, 0)

    def scatter(colb, msg):
        for r in range(MCR):
            pltpu.sync_copy(msg.at[pl.ds(r * 128, 128)], acc.at[colb.at[r]], add=True)

    fetch(rowb0, colb0, nfb0, msg0, semg0, 0)

    def pipe(i, _):
        fetch(rowb1, colb1, nfb1, msg1, semg1, 2 * i + 1)
        drain(rowb0, msg0, semg0)
        scale(nfb0, msg0)
        scatter(colb0, msg0)
        fetch(rowb0, colb0, nfb0, msg0, semg0, lax.rem(2 * i + 2, MNCHUNK))
        drain(rowb1, msg1, semg1)
        scale(nfb1, msg1)
        scatter(colb1, msg1)
        return 0

    lax.fori_loop(0, MNCHUNK // 2, pipe, 0)
    drain(rowb0, msg0, semg0)
    plsc.subcore_barrier()
    for t in range(NPT // 32):
        r0 = s * NPT + t * 32
        pltpu.sync_copy(acc.at[pl.ds(r0, 32)], zb)
        pltpu.sync_copy(zb, out_hbm.at[c, pl.ds(r0, 32)])


def _sc_msg(xw, rowp, colp, norm_flat):
    buf = lambda: [
        pltpu.VMEM((MCR, 128), jnp.int32),
        pltpu.VMEM((MCR, 128), jnp.int32),
        pltpu.VMEM((MCE,), jnp.float32),
        pltpu.VMEM((MCE, H), jnp.float32),
    ]
    f = pl.kernel(
        _msg_body,
        compiler_params=_sc_params(),
        out_type=jax.ShapeDtypeStruct((NC, NP, H), jnp.float32),
        mesh=_mesh(),
        scratch_types=buf() + buf() + [
            pltpu.VMEM((32, H), jnp.float32),
            pltpu.VMEM_SHARED((NP, H), jnp.float32),
            pltpu.VMEM_SHARED((NP, H), jnp.float32),
            pltpu.SemaphoreType.DMA,
            pltpu.SemaphoreType.DMA,
        ],
    )
    return f(xw, rowp, colp, norm_flat)


def _tc_matmul(h, w):
    def body(h_ref, w_ref, o_ref):
        o_ref[...] = jnp.dot(h_ref[...], w_ref[...],
                             preferred_element_type=jnp.float32)

    return pl.pallas_call(
        body,
        out_shape=jax.ShapeDtypeStruct((h.shape[0], w.shape[1]), jnp.float32),
    )(h, w)


def _tc_dinv(degp_t):
    def body(d_ref, o_ref):
        deg = d_ref[:, 0:1] + d_ref[:, 1:2] + 1.0
        o_ref[...] = jnp.where(deg > 0, lax.rsqrt(deg), 0.0)

    return pl.pallas_call(
        body,
        out_shape=jax.ShapeDtypeStruct((NP, 1), jnp.float32),
    )(degp_t)


def _tc_post(accp, xw, dinv_col, b2, g2, bt2, res):
    def body(*refs):
        if res is not None:
            a_ref, xw_ref, d_ref, b_ref, g_ref, bt_ref, r_ref, o_ref = refs
        else:
            a_ref, xw_ref, d_ref, b_ref, g_ref, bt_ref, o_ref = refs
        agg = a_ref[0] + a_ref[1]
        agg = agg[:N, :]
        d = d_ref[:N, :]
        xwv = xw_ref[...]
        t = agg + d * d * xwv + b_ref[...]
        t = t * (g_ref[...] * _BN_SCALE) + bt_ref[...]
        t = jnp.maximum(t, 0.0)
        if res is not None:
            t = t + r_ref[...]
        o_ref[...] = t

    args = [accp, xw, dinv_col, b2, g2, bt2]
    if res is not None:
        args.append(res)
    return pl.pallas_call(
        body,
        out_shape=jax.ShapeDtypeStruct((N, H), jnp.float32),
    )(*args)


def _tc_readout(h, batch_col, p):
    def body(h_ref, b_ref, aw1, ab1, aw2, ab2, mw1, mb1, mw2, mb2, mw3, mb3,
             o_ref):
        hv = h_ref[...]
        bcol = b_ref[...]
        gids = lax.broadcasted_iota(jnp.int32, (N, G), 1)
        M = bcol == gids
        Mf = M.astype(jnp.float32)
        dot = functools.partial(lax.dot_general,
                                dimension_numbers=(((0,), (0,)), ((), ())),
                                precision=lax.Precision.HIGHEST,
                                preferred_element_type=jnp.float32)
        ones_col = jnp.ones((N, 1), jnp.float32)
        cnt = dot(Mf, ones_col)
        x_add = dot(Mf, hv)
        x_mean = x_add / jnp.maximum(cnt, 1.0)
        a1 = jnp.tanh(jnp.dot(hv, aw1[...], preferred_element_type=jnp.float32)
                      + ab1[...])
        attn = jnp.dot(a1, aw2[...], preferred_element_type=jnp.float32) \
            + ab2[...]
        am = jnp.max(jnp.where(M, attn, -jnp.inf), axis=0, keepdims=True)
        mb = jnp.sum(jnp.where(M, am, 0.0), axis=1, keepdims=True)
        e = jnp.exp(attn - mb)
        denom = dot(Mf, e)
        den_node = jnp.dot(Mf, denom, precision=lax.Precision.HIGHEST,
                           preferred_element_type=jnp.float32)
        wnode = e / den_node
        x_attn = dot(Mf, hv * wnode)
        comb = x_mean + 0.5 * x_attn + 0.1 * x_add
        z = jnp.maximum(jnp.dot(comb, mw1[...],
                                preferred_element_type=jnp.float32)
                        + mb1[...], 0.0)
        z = jnp.maximum(jnp.dot(z, mw2[...],
                                preferred_element_type=jnp.float32)
                        + mb2[...], 0.0)
        o_ref[...] = jnp.dot(z, mw3[...],
                             preferred_element_type=jnp.float32) + mb3[...]

    return pl.pallas_call(
        body,
        out_shape=jax.ShapeDtypeStruct((G, 1), jnp.float32),
    )(h, batch_col,
      p["attnW1"], p["attnb1"].reshape(1, -1),
      p["attnW2"], p["attnb2"].reshape(1, -1),
      p["mlpW1"], p["mlpb1"].reshape(1, -1),
      p["mlpW2"], p["mlpb2"].reshape(1, -1),
      p["mlpW3"], p["mlpb3"].reshape(1, -1))


def kernel(x, edge_index, edge_weight, batch, params):
    row = edge_index[0]
    col = edge_index[1]
    padn = EPAD - E
    rowp = jnp.concatenate([row, jnp.zeros((padn,), row.dtype)]).reshape(ER, 128)
    colp = jnp.concatenate([col, jnp.zeros((padn,), col.dtype)]).reshape(ER, 128)
    wp = jnp.concatenate(
        [edge_weight, jnp.zeros((padn,), edge_weight.dtype)]).reshape(ER, 128)

    degp = _sc_deg(colp, wp)
    dinv_col = _tc_dinv(degp.T)
    norm_flat = _sc_norm(rowp, colp, wp, dinv_col.reshape(NP))

    h = None
    for i in range(LAYERS):
        inp = x if i == 0 else h
        xw = _tc_matmul(inp, params["convW"][i])
        accp = _sc_msg(xw, rowp, colp, norm_flat)
        h = _tc_post(
            accp, xw, dinv_col,
            params["convb"][i].reshape(1, H),
            params["bn_g"][i].reshape(1, H),
            params["bn_b"][i].reshape(1, H),
            res=(h if i > 0 else None),
        )

    return _tc_readout(h, batch.reshape(N, 1), params)


# speedup vs baseline: 1.2175x; 1.0624x over previous
import functools
import math

import jax
import jax.numpy as jnp
from jax import lax
from jax.experimental import pallas as pl
from jax.experimental.pallas import tpu as pltpu
from jax.experimental.pallas import tpu_sc as plsc

N = 10000
E = 320000
F_IN = 128
H = 64
G = 16
LAYERS = 4
EPS = 1e-05

NC = 2
NS = 16
LN = 16
NW = NC * NS

NP = 10240
NPT = NP // NS
EPAD = 327680
ER = EPAD // 128
RPW = ER // NW
CR = 4
CE = CR * 128
NCHUNK = RPW // CR
MCR = 2
MCE = MCR * 128
MNCHUNK = RPW // MCR

_BN_SCALE = 1.0 / math.sqrt(1.0 + EPS)


def _wid():
    return lax.axis_index("c") * NS + lax.axis_index("s")


def _mesh():
    return plsc.VectorSubcoreMesh(core_axis_name="c", subcore_axis_name="s")


def _sc_params():
    return pltpu.CompilerParams(
        needs_layout_passes=False, use_tc_tiling_on_sc=False)


def _deg_body(col_hbm, w_hbm, out_hbm, colb, wb, zb, degacc):
    c = lax.axis_index("c")
    s = lax.axis_index("s")
    wid = c * NS + s

    def zfill(i, _):
        zb[pl.ds(i * LN, LN)] = jnp.zeros((LN,), jnp.float32)
        return 0

    lax.fori_loop(0, NPT // LN, zfill, 0)
    pltpu.sync_copy(zb, degacc.at[pl.ds(s * NPT, NPT)])
    plsc.subcore_barrier()

    def chunk(i, _):
        base = wid * RPW + i * CR
        pltpu.sync_copy(col_hbm.at[pl.ds(base, CR)], colb)
        pltpu.sync_copy(w_hbm.at[pl.ds(base, CR)], wb)
        for r in range(CR):
            pltpu.sync_copy(wb.at[r], degacc.at[colb.at[r]], add=True)
        return 0

    lax.fori_loop(0, NCHUNK, chunk, 0)
    plsc.subcore_barrier()
    pltpu.sync_copy(degacc.at[pl.ds(s * NPT, NPT)], zb)
    pltpu.sync_copy(zb, out_hbm.at[c, pl.ds(s * NPT, NPT)])


def _sc_deg(colp, wp):
    f = pl.kernel(
        _deg_body,
        compiler_params=_sc_params(),
        out_type=jax.ShapeDtypeStruct((NC, NP), jnp.float32),
        mesh=_mesh(),
        scratch_types=[
            pltpu.VMEM((CR, 128), jnp.int32),
            pltpu.VMEM((CR, 128), jnp.float32),
            pltpu.VMEM((NPT,), jnp.float32),
            pltpu.VMEM_SHARED((NP,), jnp.float32),
        ],
    )
    return f(colp, wp)


def _norm_body(row_hbm, col_hbm, w_hbm, dinv_hbm, out_hbm, rb, cb, wb, nb, dinv_v):
    wid = _wid()
    pltpu.sync_copy(dinv_hbm, dinv_v)

    def chunk(i, _):
        base = wid * RPW + i * CR
        pltpu.sync_copy(row_hbm.at[pl.ds(base, CR)], rb)
        pltpu.sync_copy(col_hbm.at[pl.ds(base, CR)], cb)
        pltpu.sync_copy(w_hbm.at[pl.ds(base, CR)], wb)
        for g in range(CE // LN):
            r = g // (128 // LN)
            off = (g % (128 // LN)) * LN
            rv = rb[r, pl.ds(off, LN)]
            cv = cb[r, pl.ds(off, LN)]
            wv = wb[r, pl.ds(off, LN)]
            dr = plsc.load_gather(dinv_v, [rv])
            dc = plsc.load_gather(dinv_v, [cv])
            nb[pl.ds(g * LN, LN)] = dr * wv * dc
        pltpu.sync_copy(nb, out_hbm.at[pl.ds(base * 128, CE)])
        return 0

    lax.fori_loop(0, NCHUNK, chunk, 0)


def _sc_norm(rowp, colp, wp, dinv_flat):
    f = pl.kernel(
        _norm_body,
        compiler_params=_sc_params(),
        out_type=jax.ShapeDtypeStruct((EPAD,), jnp.float32),
        mesh=_mesh(),
        scratch_types=[
            pltpu.VMEM((CR, 128), jnp.int32),
            pltpu.VMEM((CR, 128), jnp.int32),
            pltpu.VMEM((CR, 128), jnp.float32),
            pltpu.VMEM((CE,), jnp.float32),
            pltpu.VMEM((NP,), jnp.float32),
        ],
    )
    return f(rowp, colp, wp, dinv_flat)


def _msg_body(xw_hbm, row_hbm, col_hbm, norm_hbm, out_hbm,
              rowb0, colb0, nfb0, msg0, rowb1, colb1, nfb1, msg1,
              zb, acc, xwt, semg0, semg1, semi0, semi1):
    c = lax.axis_index("c")
    s = lax.axis_index("s")
    wid = c * NS + s

    def zfill(i, _):
        for q in range(H // LN):
            zb[i, pl.ds(q * LN, LN)] = jnp.zeros((LN,), jnp.float32)
        return 0

    lax.fori_loop(0, 32, zfill, 0)
    pltpu.sync_copy(xw_hbm.at[pl.ds(s * (N // NS), N // NS)],
                    xwt.at[pl.ds(s * (N // NS), N // NS)])
    for t in range(NPT // 32):
        pltpu.sync_copy(zb, acc.at[pl.ds(s * NPT + t * 32, 32)])
    plsc.subcore_barrier()

    def fetch_idx(rowb, colb, nfb, semi, ci):
        base = wid * RPW + ci * MCR
        pltpu.async_copy(row_hbm.at[pl.ds(base, MCR)], rowb, semi)
        pltpu.async_copy(col_hbm.at[pl.ds(base, MCR)], colb, semi)
        pltpu.async_copy(norm_hbm.at[pl.ds(base * 128, MCE)], nfb, semi)

    def drain_idx(rowb, colb, nfb, semi):
        pltpu.make_async_copy(row_hbm.at[pl.ds(0, MCR)], rowb, semi).wait()
        pltpu.make_async_copy(col_hbm.at[pl.ds(0, MCR)], colb, semi).wait()
        pltpu.make_async_copy(norm_hbm.at[pl.ds(0, MCE)], nfb, semi).wait()

    def fire(rowb, msg, semg):
        for r in range(MCR):
            pltpu.async_copy(
                xwt.at[rowb.at[r]], msg.at[pl.ds(r * 128, 128)], semg)

    def drain(rowb, msg, semg):
        for r in range(MCR):
            pltpu.make_async_copy(
                xwt.at[rowb.at[r]], msg.at[pl.ds(r * 128, 128)], semg).wait()

    def scale(nfb, msg):
        def group(g, _):
            e0 = g * LN
            nvg = nfb[pl.ds(e0, LN)]
            for k in range(LN):
                nv = jnp.full((LN,), nvg[k])
                for q in range(H // LN):
                    sl = pl.ds(q * LN, LN)
                    msg[e0 + k, sl] = msg[e0 + k, sl] * nv
            return 0

        lax.fori_loop(0, MCE // LN, group, 0)

    def scatter(colb, msg):
        for r in range(MCR):
            pltpu.sync_copy(msg.at[pl.ds(r * 128, 128)], acc.at[colb.at[r]], add=True)

    fetch_idx(rowb0, colb0, nfb0, semi0, 0)
    drain_idx(rowb0, colb0, nfb0, semi0)
    fire(rowb0, msg0, semg0)
    fetch_idx(rowb1, colb1, nfb1, semi1, 1)

    def pipe(i, _):
        drain(rowb0, msg0, semg0)
        scale(nfb0, msg0)
        scatter(colb0, msg0)
        fetch_idx(rowb0, colb0, nfb0, semi0, lax.rem(2 * i + 2, MNCHUNK))
        drain_idx(rowb1, colb1, nfb1, semi1)
        fire(rowb1, msg1, semg1)
        drain(rowb1, msg1, semg1)
        scale(nfb1, msg1)
        scatter(colb1, msg1)
        fetch_idx(rowb1, colb1, nfb1, semi1, lax.rem(2 * i + 3, MNCHUNK))
        drain_idx(rowb0, colb0, nfb0, semi0)
        fire(rowb0, msg0, semg0)
        return 0

    lax.fori_loop(0, MNCHUNK // 2, pipe, 0)
    drain(rowb0, msg0, semg0)
    drain_idx(rowb1, colb1, nfb1, semi1)
    plsc.subcore_barrier()
    for t in range(NPT // 32):
        r0 = s * NPT + t * 32
        pltpu.sync_copy(acc.at[pl.ds(r0, 32)], zb)
        pltpu.sync_copy(zb, out_hbm.at[c, pl.ds(r0, 32)])


def _sc_msg(xw, rowp, colp, norm_flat):
    buf = lambda: [
        pltpu.VMEM((MCR, 128), jnp.int32),
        pltpu.VMEM((MCR, 128), jnp.int32),
        pltpu.VMEM((MCE,), jnp.float32),
        pltpu.VMEM((MCE, H), jnp.float32),
    ]
    f = pl.kernel(
        _msg_body,
        compiler_params=_sc_params(),
        out_type=jax.ShapeDtypeStruct((NC, NP, H), jnp.float32),
        mesh=_mesh(),
        scratch_types=buf() + buf() + [
            pltpu.VMEM((32, H), jnp.float32),
            pltpu.VMEM_SHARED((NP, H), jnp.float32),
            pltpu.VMEM_SHARED((NP, H), jnp.float32),
            pltpu.SemaphoreType.DMA,
            pltpu.SemaphoreType.DMA,
            pltpu.SemaphoreType.DMA,
            pltpu.SemaphoreType.DMA,
        ],
    )
    return f(xw, rowp, colp, norm_flat)


def _tc_matmul(h, w):
    def body(h_ref, w_ref, o_ref):
        o_ref[...] = jnp.dot(h_ref[...], w_ref[...],
                             preferred_element_type=jnp.float32)

    return pl.pallas_call(
        body,
        out_shape=jax.ShapeDtypeStruct((h.shape[0], w.shape[1]), jnp.float32),
    )(h, w)


def _tc_dinv(degp_t):
    def body(d_ref, o_ref):
        deg = d_ref[:, 0:1] + d_ref[:, 1:2] + 1.0
        o_ref[...] = jnp.where(deg > 0, lax.rsqrt(deg), 0.0)

    return pl.pallas_call(
        body,
        out_shape=jax.ShapeDtypeStruct((NP, 1), jnp.float32),
    )(degp_t)


def _tc_post(accp, xw, dinv_col, b2, g2, bt2, res):
    def body(*refs):
        if res is not None:
            a_ref, xw_ref, d_ref, b_ref, g_ref, bt_ref, r_ref, o_ref = refs
        else:
            a_ref, xw_ref, d_ref, b_ref, g_ref, bt_ref, o_ref = refs
        agg = a_ref[0] + a_ref[1]
        agg = agg[:N, :]
        d = d_ref[:N, :]
        xwv = xw_ref[...]
        t = agg + d * d * xwv + b_ref[...]
        t = t * (g_ref[...] * _BN_SCALE) + bt_ref[...]
        t = jnp.maximum(t, 0.0)
        if res is not None:
            t = t + r_ref[...]
        o_ref[...] = t

    args = [accp, xw, dinv_col, b2, g2, bt2]
    if res is not None:
        args.append(res)
    return pl.pallas_call(
        body,
        out_shape=jax.ShapeDtypeStruct((N, H), jnp.float32),
    )(*args)


def _tc_readout(h, batch_col, p):
    def body(h_ref, b_ref, aw1, ab1, aw2, ab2, mw1, mb1, mw2, mb2, mw3, mb3,
             o_ref):
        hv = h_ref[...]
        bcol = b_ref[...]
        gids = lax.broadcasted_iota(jnp.int32, (N, G), 1)
        M = bcol == gids
        Mf = M.astype(jnp.float32)
        dot = functools.partial(lax.dot_general,
                                dimension_numbers=(((0,), (0,)), ((), ())),
                                precision=lax.Precision.HIGHEST,
                                preferred_element_type=jnp.float32)
        ones_col = jnp.ones((N, 1), jnp.float32)
        cnt = dot(Mf, ones_col)
        x_add = dot(Mf, hv)
        x_mean = x_add / jnp.maximum(cnt, 1.0)
        a1 = jnp.tanh(jnp.dot(hv, aw1[...], preferred_element_type=jnp.float32)
                      + ab1[...])
        attn = jnp.dot(a1, aw2[...], preferred_element_type=jnp.float32) \
            + ab2[...]
        am = jnp.max(jnp.where(M, attn, -jnp.inf), axis=0, keepdims=True)
        mb = jnp.sum(jnp.where(M, am, 0.0), axis=1, keepdims=True)
        e = jnp.exp(attn - mb)
        denom = dot(Mf, e)
        den_node = jnp.dot(Mf, denom, precision=lax.Precision.HIGHEST,
                           preferred_element_type=jnp.float32)
        wnode = e / den_node
        x_attn = dot(Mf, hv * wnode)
        comb = x_mean + 0.5 * x_attn + 0.1 * x_add
        z = jnp.maximum(jnp.dot(comb, mw1[...],
                                preferred_element_type=jnp.float32)
                        + mb1[...], 0.0)
        z = jnp.maximum(jnp.dot(z, mw2[...],
                                preferred_element_type=jnp.float32)
                        + mb2[...], 0.0)
        o_ref[...] = jnp.dot(z, mw3[...],
                             preferred_element_type=jnp.float32) + mb3[...]

    return pl.pallas_call(
        body,
        out_shape=jax.ShapeDtypeStruct((G, 1), jnp.float32),
    )(h, batch_col,
      p["attnW1"], p["attnb1"].reshape(1, -1),
      p["attnW2"], p["attnb2"].reshape(1, -1),
      p["mlpW1"], p["mlpb1"].reshape(1, -1),
      p["mlpW2"], p["mlpb2"].reshape(1, -1),
      p["mlpW3"], p["mlpb3"].reshape(1, -1))


def kernel(x, edge_index, edge_weight, batch, params):
    row = edge_index[0]
    col = edge_index[1]
    padn = EPAD - E
    rowp = jnp.concatenate([row, jnp.zeros((padn,), row.dtype)]).reshape(ER, 128)
    colp = jnp.concatenate([col, jnp.zeros((padn,), col.dtype)]).reshape(ER, 128)
    wp = jnp.concatenate(
        [edge_weight, jnp.zeros((padn,), edge_weight.dtype)]).reshape(ER, 128)

    degp = _sc_deg(colp, wp)
    dinv_col = _tc_dinv(degp.T)
    norm_flat = _sc_norm(rowp, colp, wp, dinv_col.reshape(NP))

    h = None
    for i in range(LAYERS):
        inp = x if i == 0 else h
        xw = _tc_matmul(inp, params["convW"][i])
        accp = _sc_msg(xw, rowp, colp, norm_flat)
        h = _tc_post(
            accp, xw, dinv_col,
            params["convb"][i].reshape(1, H),
            params["bn_g"][i].reshape(1, H),
            params["bn_b"][i].reshape(1, H),
            res=(h if i > 0 else None),
        )

    return _tc_readout(h, batch.reshape(N, 1), params)


# speedup vs baseline: 1.8698x; 1.5357x over previous
import functools
import math

import jax
import jax.numpy as jnp
from jax import lax
from jax.experimental import pallas as pl
from jax.experimental.pallas import tpu as pltpu
from jax.experimental.pallas import tpu_sc as plsc

N = 10000
E = 320000
F_IN = 128
H = 64
G = 16
LAYERS = 4
EPS = 1e-05

NC = 2
NS = 16
LN = 16
NW = NC * NS

NP = 10240
NPT = NP // NS
EPAD = 327680
ER = EPAD // 128
RPW = ER // NW
CR = 4
CE = CR * 128
NCHUNK = RPW // CR
MCR = 2
MCE = MCR * 128
MNCHUNK = RPW // MCR

_BN_SCALE = 1.0 / math.sqrt(1.0 + EPS)


def _wid():
    return lax.axis_index("c") * NS + lax.axis_index("s")


def _mesh():
    return plsc.VectorSubcoreMesh(core_axis_name="c", subcore_axis_name="s")


def _sc_params():
    return pltpu.CompilerParams(
        needs_layout_passes=False, use_tc_tiling_on_sc=False)


def _deg_body(col_hbm, w_hbm, out_hbm, colb, wb, zb, degacc):
    c = lax.axis_index("c")
    s = lax.axis_index("s")
    wid = c * NS + s

    def zfill(i, _):
        zb[pl.ds(i * LN, LN)] = jnp.zeros((LN,), jnp.float32)
        return 0

    lax.fori_loop(0, NPT // LN, zfill, 0)
    pltpu.sync_copy(zb, degacc.at[pl.ds(s * NPT, NPT)])
    plsc.subcore_barrier()

    def chunk(i, _):
        base = wid * RPW + i * CR
        pltpu.sync_copy(col_hbm.at[pl.ds(base, CR)], colb)
        pltpu.sync_copy(w_hbm.at[pl.ds(base, CR)], wb)
        for r in range(CR):
            pltpu.sync_copy(wb.at[r], degacc.at[colb.at[r]], add=True)
        return 0

    lax.fori_loop(0, NCHUNK, chunk, 0)
    plsc.subcore_barrier()
    pltpu.sync_copy(degacc.at[pl.ds(s * NPT, NPT)], zb)
    pltpu.sync_copy(zb, out_hbm.at[c, pl.ds(s * NPT, NPT)])


def _sc_deg(colp, wp):
    f = pl.kernel(
        _deg_body,
        compiler_params=_sc_params(),
        out_type=jax.ShapeDtypeStruct((NC, NP), jnp.float32),
        mesh=_mesh(),
        scratch_types=[
            pltpu.VMEM((CR, 128), jnp.int32),
            pltpu.VMEM((CR, 128), jnp.float32),
            pltpu.VMEM((NPT,), jnp.float32),
            pltpu.VMEM_SHARED((NP,), jnp.float32),
        ],
    )
    return f(colp, wp)


def _norm_body(row_hbm, col_hbm, w_hbm, dinv_hbm, out_hbm, rb, cb, wb, nb, dinv_v):
    wid = _wid()
    pltpu.sync_copy(dinv_hbm, dinv_v)

    def chunk(i, _):
        base = wid * RPW + i * CR
        pltpu.sync_copy(row_hbm.at[pl.ds(base, CR)], rb)
        pltpu.sync_copy(col_hbm.at[pl.ds(base, CR)], cb)
        pltpu.sync_copy(w_hbm.at[pl.ds(base, CR)], wb)
        for g in range(CE // LN):
            r = g // (128 // LN)
            off = (g % (128 // LN)) * LN
            rv = rb[r, pl.ds(off, LN)]
            cv = cb[r, pl.ds(off, LN)]
            wv = wb[r, pl.ds(off, LN)]
            dr = plsc.load_gather(dinv_v, [rv])
            dc = plsc.load_gather(dinv_v, [cv])
            nb[pl.ds(g * LN, LN)] = dr * wv * dc
        pltpu.sync_copy(nb, out_hbm.at[pl.ds(base * 128, CE)])
        return 0

    lax.fori_loop(0, NCHUNK, chunk, 0)


def _sc_norm(rowp, colp, wp, dinv_flat):
    f = pl.kernel(
        _norm_body,
        compiler_params=_sc_params(),
        out_type=jax.ShapeDtypeStruct((EPAD,), jnp.float32),
        mesh=_mesh(),
        scratch_types=[
            pltpu.VMEM((CR, 128), jnp.int32),
            pltpu.VMEM((CR, 128), jnp.int32),
            pltpu.VMEM((CR, 128), jnp.float32),
            pltpu.VMEM((CE,), jnp.float32),
            pltpu.VMEM((NP,), jnp.float32),
        ],
    )
    return f(rowp, colp, wp, dinv_flat)


def _msg_body(xw_hbm, row_hbm, col_hbm, norm_hbm, zeros_hbm, out_hbm,
              rowb0, colb0, nfb0, msg0, rowb1, colb1, nfb1, msg1,
              acc, xwt, semg0, semg1, semi0, semi1):
    c = lax.axis_index("c")
    s = lax.axis_index("s")
    wid = c * NS + s

    pltpu.sync_copy(xw_hbm.at[pl.ds(s * (N // NS), N // NS)],
                    xwt.at[pl.ds(s * (N // NS), N // NS)])
    pltpu.sync_copy(zeros_hbm, acc.at[pl.ds(s * NPT, NPT)])
    plsc.subcore_barrier()

    def fetch_idx(rowb, colb, nfb, semi, ci):
        base = wid * RPW + ci * MCR
        pltpu.async_copy(row_hbm.at[pl.ds(base, MCR)], rowb, semi)
        pltpu.async_copy(col_hbm.at[pl.ds(base, MCR)], colb, semi)
        pltpu.async_copy(norm_hbm.at[pl.ds(base * 128, MCE)], nfb, semi)

    def drain_idx(rowb, colb, nfb, semi):
        pltpu.make_async_copy(row_hbm.at[pl.ds(0, MCR)], rowb, semi).wait()
        pltpu.make_async_copy(col_hbm.at[pl.ds(0, MCR)], colb, semi).wait()
        pltpu.make_async_copy(norm_hbm.at[pl.ds(0, MCE)], nfb, semi).wait()

    def fire(rowb, msg, semg):
        for r in range(MCR):
            pltpu.async_copy(
                xwt.at[rowb.at[r]], msg.at[pl.ds(r * 128, 128)], semg)

    def drain(rowb, msg, semg):
        for r in range(MCR):
            pltpu.make_async_copy(
                xwt.at[rowb.at[r]], msg.at[pl.ds(r * 128, 128)], semg).wait()

    def scale(nfb, msg):
        def group(g, _):
            e0 = g * LN
            nvg = nfb[pl.ds(e0, LN)]
            for k in range(LN):
                nv = jnp.full((LN,), nvg[k])
                for q in range(H // LN):
                    sl = pl.ds(q * LN, LN)
                    msg[e0 + k, sl] = msg[e0 + k, sl] * nv
            return 0

        lax.fori_loop(0, MCE // LN, group, 0)

    def scatter(colb, msg):
        for r in range(MCR):
            pltpu.sync_copy(msg.at[pl.ds(r * 128, 128)], acc.at[colb.at[r]], add=True)

    fetch_idx(rowb0, colb0, nfb0, semi0, 0)
    drain_idx(rowb0, colb0, nfb0, semi0)
    fire(rowb0, msg0, semg0)
    fetch_idx(rowb1, colb1, nfb1, semi1, 1)

    def pipe(i, _):
        drain(rowb0, msg0, semg0)
        scale(nfb0, msg0)
        scatter(colb0, msg0)
        fetch_idx(rowb0, colb0, nfb0, semi0, lax.rem(2 * i + 2, MNCHUNK))
        drain_idx(rowb1, colb1, nfb1, semi1)
        fire(rowb1, msg1, semg1)
        drain(rowb1, msg1, semg1)
        scale(nfb1, msg1)
        scatter(colb1, msg1)
        fetch_idx(rowb1, colb1, nfb1, semi1, lax.rem(2 * i + 3, MNCHUNK))
        drain_idx(rowb0, colb0, nfb0, semi0)
        fire(rowb0, msg0, semg0)
        return 0

    lax.fori_loop(0, MNCHUNK // 2, pipe, 0)
    drain(rowb0, msg0, semg0)
    drain_idx(rowb1, colb1, nfb1, semi1)
    plsc.subcore_barrier()
    pltpu.sync_copy(acc.at[pl.ds(s * NPT, NPT)],
                    out_hbm.at[c, pl.ds(s * NPT, NPT)])


def _sc_msg(xw, rowp, colp, norm_flat, zrows):
    buf = lambda: [
        pltpu.VMEM((MCR, 128), jnp.int32),
        pltpu.VMEM((MCR, 128), jnp.int32),
        pltpu.VMEM((MCE,), jnp.float32),
        pltpu.VMEM((MCE, H), jnp.float32),
    ]
    f = pl.kernel(
        _msg_body,
        compiler_params=_sc_params(),
        out_type=jax.ShapeDtypeStruct((NC, NP, H), jnp.float32),
        mesh=_mesh(),
        scratch_types=buf() + buf() + [
            pltpu.VMEM_SHARED((NP, H), jnp.float32),
            pltpu.VMEM_SHARED((NP, H), jnp.float32),
            pltpu.SemaphoreType.DMA,
            pltpu.SemaphoreType.DMA,
            pltpu.SemaphoreType.DMA,
            pltpu.SemaphoreType.DMA,
        ],
    )
    return f(xw, rowp, colp, norm_flat, zrows)


def _tc_matmul(h, w):
    def body(h_ref, w_ref, o_ref):
        o_ref[...] = jnp.dot(h_ref[...], w_ref[...],
                             preferred_element_type=jnp.float32)

    return pl.pallas_call(
        body,
        out_shape=jax.ShapeDtypeStruct((h.shape[0], w.shape[1]), jnp.float32),
    )(h, w)


def _tc_dinv(degp_t):
    def body(d_ref, o_ref):
        deg = d_ref[:, 0:1] + d_ref[:, 1:2] + 1.0
        o_ref[...] = jnp.where(deg > 0, lax.rsqrt(deg), 0.0)

    return pl.pallas_call(
        body,
        out_shape=jax.ShapeDtypeStruct((NP, 1), jnp.float32),
    )(degp_t)


def _tc_post_mm(accp, xw, dinv_col, b2, g2, bt2, res, w_next):
    def body(*refs):
        if res is not None:
            a_ref, xw_ref, d_ref, b_ref, g_ref, bt_ref, r_ref, w_ref, o_ref, xo_ref = refs
        else:
            a_ref, xw_ref, d_ref, b_ref, g_ref, bt_ref, w_ref, o_ref, xo_ref = refs
        agg = a_ref[0] + a_ref[1]
        agg = agg[:N, :]
        d = d_ref[:N, :]
        xwv = xw_ref[...]
        t = agg + d * d * xwv + b_ref[...]
        t = t * (g_ref[...] * _BN_SCALE) + bt_ref[...]
        t = jnp.maximum(t, 0.0)
        if res is not None:
            t = t + r_ref[...]
        o_ref[...] = t
        xo_ref[...] = jnp.dot(t, w_ref[...], preferred_element_type=jnp.float32)

    args = [accp, xw, dinv_col, b2, g2, bt2]
    if res is not None:
        args.append(res)
    args.append(w_next)
    return pl.pallas_call(
        body,
        out_shape=(jax.ShapeDtypeStruct((N, H), jnp.float32),
                   jax.ShapeDtypeStruct((N, H), jnp.float32)),
    )(*args)


def _tc_post(accp, xw, dinv_col, b2, g2, bt2, res):
    def body(*refs):
        if res is not None:
            a_ref, xw_ref, d_ref, b_ref, g_ref, bt_ref, r_ref, o_ref = refs
        else:
            a_ref, xw_ref, d_ref, b_ref, g_ref, bt_ref, o_ref = refs
        agg = a_ref[0] + a_ref[1]
        agg = agg[:N, :]
        d = d_ref[:N, :]
        xwv = xw_ref[...]
        t = agg + d * d * xwv + b_ref[...]
        t = t * (g_ref[...] * _BN_SCALE) + bt_ref[...]
        t = jnp.maximum(t, 0.0)
        if res is not None:
            t = t + r_ref[...]
        o_ref[...] = t

    args = [accp, xw, dinv_col, b2, g2, bt2]
    if res is not None:
        args.append(res)
    return pl.pallas_call(
        body,
        out_shape=jax.ShapeDtypeStruct((N, H), jnp.float32),
    )(*args)


def _tc_readout(h, batch_col, p):
    def body(h_ref, b_ref, aw1, ab1, aw2, ab2, mw1, mb1, mw2, mb2, mw3, mb3,
             o_ref):
        hv = h_ref[...]
        bcol = b_ref[...]
        gids = lax.broadcasted_iota(jnp.int32, (N, G), 1)
        M = bcol == gids
        Mf = M.astype(jnp.float32)
        dot = functools.partial(lax.dot_general,
                                dimension_numbers=(((0,), (0,)), ((), ())),
                                precision=lax.Precision.HIGHEST,
                                preferred_element_type=jnp.float32)
        ones_col = jnp.ones((N, 1), jnp.float32)
        cnt = dot(Mf, ones_col)
        x_add = dot(Mf, hv)
        x_mean = x_add / jnp.maximum(cnt, 1.0)
        a1 = jnp.tanh(jnp.dot(hv, aw1[...], preferred_element_type=jnp.float32)
                      + ab1[...])
        attn = jnp.dot(a1, aw2[...], preferred_element_type=jnp.float32) \
            + ab2[...]
        am = jnp.max(jnp.where(M, attn, -jnp.inf), axis=0, keepdims=True)
        mb = jnp.sum(jnp.where(M, am, 0.0), axis=1, keepdims=True)
        e = jnp.exp(attn - mb)
        denom = dot(Mf, e)
        den_node = jnp.dot(Mf, denom, precision=lax.Precision.HIGHEST,
                           preferred_element_type=jnp.float32)
        wnode = e / den_node
        x_attn = dot(Mf, hv * wnode)
        comb = x_mean + 0.5 * x_attn + 0.1 * x_add
        z = jnp.maximum(jnp.dot(comb, mw1[...],
                                preferred_element_type=jnp.float32)
                        + mb1[...], 0.0)
        z = jnp.maximum(jnp.dot(z, mw2[...],
                                preferred_element_type=jnp.float32)
                        + mb2[...], 0.0)
        o_ref[...] = jnp.dot(z, mw3[...],
                             preferred_element_type=jnp.float32) + mb3[...]

    return pl.pallas_call(
        body,
        out_shape=jax.ShapeDtypeStruct((G, 1), jnp.float32),
    )(h, batch_col,
      p["attnW1"], p["attnb1"].reshape(1, -1),
      p["attnW2"], p["attnb2"].reshape(1, -1),
      p["mlpW1"], p["mlpb1"].reshape(1, -1),
      p["mlpW2"], p["mlpb2"].reshape(1, -1),
      p["mlpW3"], p["mlpb3"].reshape(1, -1))


def kernel(x, edge_index, edge_weight, batch, params):
    row = edge_index[0]
    col = edge_index[1]
    padn = EPAD - E
    rowp = jnp.concatenate([row, jnp.zeros((padn,), row.dtype)]).reshape(ER, 128)
    colp = jnp.concatenate([col, jnp.zeros((padn,), col.dtype)]).reshape(ER, 128)
    wp = jnp.concatenate(
        [edge_weight, jnp.zeros((padn,), edge_weight.dtype)]).reshape(ER, 128)

    degp = _sc_deg(colp, wp)
    dinv_col = _tc_dinv(degp.T)
    norm_flat = _sc_norm(rowp, colp, wp, dinv_col.reshape(NP))

    zrows = jnp.zeros((NPT, H), jnp.float32)
    h = None
    xw = _tc_matmul(x, params["convW"][0])
    for i in range(LAYERS):
        accp = _sc_msg(xw, rowp, colp, norm_flat, zrows)
        pargs = (accp, xw, dinv_col,
                 params["convb"][i].reshape(1, H),
                 params["bn_g"][i].reshape(1, H),
                 params["bn_b"][i].reshape(1, H),
                 h if i > 0 else None)
        if i + 1 < LAYERS:
            h, xw = _tc_post_mm(*pargs, params["convW"][i + 1])
        else:
            h = _tc_post(*pargs)

    return _tc_readout(h, batch.reshape(N, 1), params)
